# baseline (speedup 1.0000x reference)
import jax
import jax.numpy as jnp
from jax import lax
from jax.experimental import pallas as pl
from jax.experimental.pallas import tpu as pltpu
from jax.experimental.pallas import tpu_sc as plsc

B, S, D = 4, 8192, 768
NC, NS, L = 2, 16, 16
SEG = S // 8
K = 64
NJ = D // L
IDX_CAP = SEG + K
CROW = 128


def _pool_body(feat_hbm, mask_hbm, out_hbm,
               mask_v, idx_v, gbuf, acc_v, r0row_v, cnt_v,
               part_v, cbuf_v, res_v, shared_part, shared_cnt, sem):
    c = lax.axis_index("c")
    s = lax.axis_index("s")
    b = c * 2 + s // 8
    base = b * S + (s % 8) * SEG

    pltpu.sync_copy(mask_hbm.at[pl.ds(base, SEG)], mask_v)

    r0v = jnp.full((L,), 0, dtype=jnp.int32) + base
    for t in range(IDX_CAP // L):
        idx_v[pl.ds(t * L, L)] = r0v

    lane = lax.broadcasted_iota(jnp.int32, (L,), 0)

    def mbody(i, off):
        mv = mask_v[pl.ds(i * L, L)]
        mb = mv != 0
        mi = mb.astype(jnp.int32)
        pos = off + plsc.cumsum(mi) - 1
        plsc.store_scatter(idx_v, [pos], base + i * L + lane, mask=mb)
        return off + jnp.sum(mi)

    n = lax.fori_loop(0, SEG // L, mbody, jnp.int32(0))

    zero = jnp.zeros((L,), jnp.float32)
    for j in range(NJ):
        acc_v[pl.ds(j * L, L)] = zero
    pltpu.sync_copy(feat_hbm.at[base], r0row_v)

    nrounds = (n + (K - 1)) // K

    def rbody(r, _):
        pltpu.async_copy(feat_hbm.at[idx_v.at[pl.ds(r * K, K)]], gbuf,
                         sem).wait()

        def jbody(j, _):
            dj = pl.ds(j * L, L)
            p0 = gbuf[0, dj]
            p1 = gbuf[1, dj]
            p2 = gbuf[2, dj]
            p3 = gbuf[3, dj]
            for k in range(4, K, 4):
                p0 = p0 + gbuf[k, dj]
                p1 = p1 + gbuf[k + 1, dj]
                p2 = p2 + gbuf[k + 2, dj]
                p3 = p3 + gbuf[k + 3, dj]
            plsc.addupdate(acc_v.at[dj], (p0 + p1) + (p2 + p3))
            return 0

        lax.fori_loop(0, NJ, jbody, 0)
        return 0

    lax.fori_loop(0, nrounds, rbody, 0)

    pad = (nrounds * K - n).astype(jnp.float32)
    for j in range(NJ):
        dj = pl.ds(j * L, L)
        acc_v[dj] = acc_v[dj] - pad * r0row_v[dj]

    cnt_v[pl.ds(0, L)] = (jnp.full((L,), 0, jnp.int32) + n).astype(jnp.float32)
    pltpu.sync_copy(acc_v, shared_part.at[pl.ds(s * D, D)])
    pltpu.sync_copy(cnt_v, shared_cnt.at[pl.ds(s * CROW, CROW)])
    plsc.subcore_barrier()

    @pl.when(s % 8 == 0)
    def _leader():
        pltpu.sync_copy(shared_part.at[pl.ds(s * D, 8 * D)], part_v)
        pltpu.sync_copy(shared_cnt.at[pl.ds(s * CROW, 8 * CROW)], cbuf_v)
        tot = cbuf_v[pl.ds(0, L)]
        for w in range(1, 8):
            tot = tot + cbuf_v[pl.ds(w * CROW, L)]
        denom = jnp.maximum(tot, 1.0)
        for j in range(NJ):
            tv = part_v[pl.ds(j * L, L)]
            for w in range(1, 8):
                tv = tv + part_v[pl.ds(w * D + j * L, L)]
            res_v[pl.ds(j * L, L)] = tv / denom
        pltpu.sync_copy(res_v, out_hbm.at[pl.ds(b * D, D)])


@jax.jit
def _pool(feat2d, mask_i32):
    kern = pl.kernel(
        _pool_body,
        out_type=jax.ShapeDtypeStruct((B * D,), jnp.float32),
        mesh=plsc.VectorSubcoreMesh(core_axis_name="c", subcore_axis_name="s"),
        scratch_types=[
            pltpu.VMEM((SEG,), jnp.int32),
            pltpu.VMEM((IDX_CAP,), jnp.int32),
            pltpu.VMEM((K, D), jnp.float32),
            pltpu.VMEM((D,), jnp.float32),
            pltpu.VMEM((D,), jnp.float32),
            pltpu.VMEM((CROW,), jnp.float32),
            pltpu.VMEM((8 * D,), jnp.float32),
            pltpu.VMEM((8 * CROW,), jnp.float32),
            pltpu.VMEM((D,), jnp.float32),
            pltpu.VMEM_SHARED((NS * D,), jnp.float32),
            pltpu.VMEM_SHARED((NS * CROW,), jnp.float32),
            pltpu.SemaphoreType.DMA,
        ],
        compiler_params=pltpu.CompilerParams(needs_layout_passes=False),
    )
    return kern(feat2d, mask_i32)


def kernel(features, mask):
    feat2d = features.reshape(B * S, D)
    mask_i32 = mask.reshape(-1).astype(jnp.int32)
    return _pool(feat2d, mask_i32).reshape(B, D)


# speedup vs baseline: 1.2969x; 1.2969x over previous
import jax
import jax.numpy as jnp
from jax import lax
from jax.experimental import pallas as pl
from jax.experimental.pallas import tpu as pltpu
from jax.experimental.pallas import tpu_sc as plsc

B, S, D = 4, 8192, 768
NC, NS, L = 2, 16, 16
SEG = S // 8
K = 64
NJ = D // L
IDX_CAP = SEG + K
CROW = 128


def _pool_body(feat_hbm, mask_hbm, out_hbm,
               mask_v, idx_v, gbuf, gbuf2, acc_v, r0row_v, cnt_v,
               part_v, cbuf_v, res_v, shared_part, shared_cnt, sem, sem2):
    c = lax.axis_index("c")
    s = lax.axis_index("s")
    b = c * 2 + s // 8
    base = b * S + (s % 8) * SEG

    pltpu.sync_copy(mask_hbm.at[pl.ds(base, SEG)], mask_v)

    r0v = jnp.full((L,), 0, dtype=jnp.int32) + base
    for t in range(IDX_CAP // L):
        idx_v[pl.ds(t * L, L)] = r0v

    lane = lax.broadcasted_iota(jnp.int32, (L,), 0)

    def mbody(i, off):
        mv = mask_v[pl.ds(i * L, L)]
        mb = mv != 0
        mi = mb.astype(jnp.int32)
        pos = off + plsc.cumsum(mi) - 1
        plsc.store_scatter(idx_v, [pos], base + i * L + lane, mask=mb)
        return off + jnp.sum(mi)

    n = lax.fori_loop(0, SEG // L, mbody, jnp.int32(0))

    zero = jnp.zeros((L,), jnp.float32)
    for j in range(NJ):
        acc_v[pl.ds(j * L, L)] = zero
    pltpu.sync_copy(feat_hbm.at[base], r0row_v)

    nrounds = (n + (K - 1)) // K

    bufs = (gbuf, gbuf2)
    sems = (sem, sem2)

    def start(r, buf, sm):
        @pl.when(r < nrounds)
        def _():
            pltpu.async_copy(feat_hbm.at[idx_v.at[pl.ds(r * K, K)]], buf, sm)

    def finish(r, buf, sm):
        @pl.when(r < nrounds)
        def _():
            pltpu.make_async_copy(feat_hbm.at[idx_v.at[pl.ds(r * K, K)]],
                                  buf, sm).wait()

            def jbody(j, _):
                dj = pl.ds(j * L, L)
                p0 = buf[0, dj]
                p1 = buf[1, dj]
                p2 = buf[2, dj]
                p3 = buf[3, dj]
                for k in range(4, K, 4):
                    p0 = p0 + buf[k, dj]
                    p1 = p1 + buf[k + 1, dj]
                    p2 = p2 + buf[k + 2, dj]
                    p3 = p3 + buf[k + 3, dj]
                plsc.addupdate(acc_v.at[dj], (p0 + p1) + (p2 + p3))
                return 0

            lax.fori_loop(0, NJ, jbody, 0)

    start(jnp.int32(0), bufs[0], sems[0])
    start(jnp.int32(1), bufs[1], sems[1])

    def pbody(r2, _):
        ra = 2 * r2
        finish(ra, bufs[0], sems[0])
        start(ra + 2, bufs[0], sems[0])
        finish(ra + 1, bufs[1], sems[1])
        start(ra + 3, bufs[1], sems[1])
        return 0

    lax.fori_loop(0, (nrounds + 1) // 2, pbody, 0)

    pad = (nrounds * K - n).astype(jnp.float32)
    for j in range(NJ):
        dj = pl.ds(j * L, L)
        acc_v[dj] = acc_v[dj] - pad * r0row_v[dj]

    cnt_v[pl.ds(0, L)] = (jnp.full((L,), 0, jnp.int32) + n).astype(jnp.float32)
    pltpu.sync_copy(acc_v, shared_part.at[pl.ds(s * D, D)])
    pltpu.sync_copy(cnt_v, shared_cnt.at[pl.ds(s * CROW, CROW)])
    plsc.subcore_barrier()

    @pl.when(s % 8 == 0)
    def _leader():
        pltpu.sync_copy(shared_part.at[pl.ds(s * D, 8 * D)], part_v)
        pltpu.sync_copy(shared_cnt.at[pl.ds(s * CROW, 8 * CROW)], cbuf_v)
        tot = cbuf_v[pl.ds(0, L)]
        for w in range(1, 8):
            tot = tot + cbuf_v[pl.ds(w * CROW, L)]
        denom = jnp.maximum(tot, 1.0)
        for j in range(NJ):
            tv = part_v[pl.ds(j * L, L)]
            for w in range(1, 8):
                tv = tv + part_v[pl.ds(w * D + j * L, L)]
            res_v[pl.ds(j * L, L)] = tv / denom
        pltpu.sync_copy(res_v, out_hbm.at[pl.ds(b * D, D)])


@jax.jit
def _pool(feat2d, mask_i32):
    kern = pl.kernel(
        _pool_body,
        out_type=jax.ShapeDtypeStruct((B * D,), jnp.float32),
        mesh=plsc.VectorSubcoreMesh(core_axis_name="c", subcore_axis_name="s"),
        scratch_types=[
            pltpu.VMEM((SEG,), jnp.int32),
            pltpu.VMEM((IDX_CAP,), jnp.int32),
            pltpu.VMEM((K, D), jnp.float32),
            pltpu.VMEM((K, D), jnp.float32),
            pltpu.VMEM((D,), jnp.float32),
            pltpu.VMEM((D,), jnp.float32),
            pltpu.VMEM((CROW,), jnp.float32),
            pltpu.VMEM((8 * D,), jnp.float32),
            pltpu.VMEM((8 * CROW,), jnp.float32),
            pltpu.VMEM((D,), jnp.float32),
            pltpu.VMEM_SHARED((NS * D,), jnp.float32),
            pltpu.VMEM_SHARED((NS * CROW,), jnp.float32),
            pltpu.SemaphoreType.DMA,
            pltpu.SemaphoreType.DMA,
        ],
        compiler_params=pltpu.CompilerParams(needs_layout_passes=False),
    )
    return kern(feat2d, mask_i32)


def kernel(features, mask):
    feat2d = features.reshape(B * S, D)
    mask_i32 = mask.reshape(-1).astype(jnp.int32)
    return _pool(feat2d, mask_i32).reshape(B, D)


# speedup vs baseline: 1.4187x; 1.0939x over previous
import jax
import jax.numpy as jnp
from jax import lax
from jax.experimental import pallas as pl
from jax.experimental.pallas import tpu as pltpu
from jax.experimental.pallas import tpu_sc as plsc

B, S, D = 4, 8192, 768
NC, NS, L = 2, 16, 16
SEG = S // 8
K = 32
NBUF = 4
NJ = D // L
IDX_CAP = SEG + K
CROW = 128


def _pool_body(feat_hbm, mask_hbm, out_hbm,
               mask_v, idx_v, gbuf, gbuf2, gbuf3, gbuf4, acc_v, r0row_v,
               cnt_v, part_v, cbuf_v, res_v, shared_part, shared_cnt,
               sem, sem2, sem3, sem4):
    c = lax.axis_index("c")
    s = lax.axis_index("s")
    b = c * 2 + s // 8
    base = b * S + (s % 8) * SEG

    pltpu.sync_copy(mask_hbm.at[pl.ds(base, SEG)], mask_v)

    r0v = jnp.full((L,), 0, dtype=jnp.int32) + base
    for t in range(IDX_CAP // L):
        idx_v[pl.ds(t * L, L)] = r0v

    lane = lax.broadcasted_iota(jnp.int32, (L,), 0)

    def mbody(i, off):
        mv = mask_v[pl.ds(i * L, L)]
        mb = mv != 0
        mi = mb.astype(jnp.int32)
        pos = off + plsc.cumsum(mi) - 1
        plsc.store_scatter(idx_v, [pos], base + i * L + lane, mask=mb)
        return off + jnp.sum(mi)

    n = lax.fori_loop(0, SEG // L, mbody, jnp.int32(0))

    zero = jnp.zeros((L,), jnp.float32)
    for j in range(NJ):
        acc_v[pl.ds(j * L, L)] = zero
    pltpu.sync_copy(feat_hbm.at[base], r0row_v)

    nrounds = (n + (K - 1)) // K

    bufs = (gbuf, gbuf2, gbuf3, gbuf4)
    sems = (sem, sem2, sem3, sem4)

    def start(r, buf, sm):
        @pl.when(r < nrounds)
        def _():
            pltpu.async_copy(feat_hbm.at[idx_v.at[pl.ds(r * K, K)]], buf, sm)

    def finish(r, buf, sm):
        @pl.when(r < nrounds)
        def _():
            pltpu.make_async_copy(feat_hbm.at[idx_v.at[pl.ds(r * K, K)]],
                                  buf, sm).wait()

            def jbody(j, _):
                dj = pl.ds(j * L, L)
                p0 = buf[0, dj]
                p1 = buf[1, dj]
                p2 = buf[2, dj]
                p3 = buf[3, dj]
                for k in range(4, K, 4):
                    p0 = p0 + buf[k, dj]
                    p1 = p1 + buf[k + 1, dj]
                    p2 = p2 + buf[k + 2, dj]
                    p3 = p3 + buf[k + 3, dj]
                plsc.addupdate(acc_v.at[dj], (p0 + p1) + (p2 + p3))
                return 0

            lax.fori_loop(0, NJ, jbody, 0)

    for t in range(NBUF):
        start(jnp.int32(t), bufs[t], sems[t])

    def pbody(r2, _):
        ra = NBUF * r2
        for t in range(NBUF):
            finish(ra + t, bufs[t], sems[t])
            start(ra + t + NBUF, bufs[t], sems[t])
        return 0

    lax.fori_loop(0, (nrounds + NBUF - 1) // NBUF, pbody, 0)

    pad = (nrounds * K - n).astype(jnp.float32)
    for j in range(NJ):
        dj = pl.ds(j * L, L)
        acc_v[dj] = acc_v[dj] - pad * r0row_v[dj]

    cnt_v[pl.ds(0, L)] = (jnp.full((L,), 0, jnp.int32) + n).astype(jnp.float32)
    pltpu.sync_copy(acc_v, shared_part.at[pl.ds(s * D, D)])
    pltpu.sync_copy(cnt_v, shared_cnt.at[pl.ds(s * CROW, CROW)])
    plsc.subcore_barrier()

    @pl.when(s % 8 == 0)
    def _leader():
        pltpu.sync_copy(shared_part.at[pl.ds(s * D, 8 * D)], part_v)
        pltpu.sync_copy(shared_cnt.at[pl.ds(s * CROW, 8 * CROW)], cbuf_v)
        tot = cbuf_v[pl.ds(0, L)]
        for w in range(1, 8):
            tot = tot + cbuf_v[pl.ds(w * CROW, L)]
        denom = jnp.maximum(tot, 1.0)
        for j in range(NJ):
            tv = part_v[pl.ds(j * L, L)]
            for w in range(1, 8):
                tv = tv + part_v[pl.ds(w * D + j * L, L)]
            res_v[pl.ds(j * L, L)] = tv / denom
        pltpu.sync_copy(res_v, out_hbm.at[pl.ds(b * D, D)])


@jax.jit
def _pool(feat2d, mask_i32):
    kern = pl.kernel(
        _pool_body,
        out_type=jax.ShapeDtypeStruct((B * D,), jnp.float32),
        mesh=plsc.VectorSubcoreMesh(core_axis_name="c", subcore_axis_name="s"),
        scratch_types=[
            pltpu.VMEM((SEG,), jnp.int32),
            pltpu.VMEM((IDX_CAP,), jnp.int32),
            pltpu.VMEM((K, D), jnp.float32),
            pltpu.VMEM((K, D), jnp.float32),
            pltpu.VMEM((K, D), jnp.float32),
            pltpu.VMEM((K, D), jnp.float32),
            pltpu.VMEM((D,), jnp.float32),
            pltpu.VMEM((D,), jnp.float32),
            pltpu.VMEM((CROW,), jnp.float32),
            pltpu.VMEM((8 * D,), jnp.float32),
            pltpu.VMEM((8 * CROW,), jnp.float32),
            pltpu.VMEM((D,), jnp.float32),
            pltpu.VMEM_SHARED((NS * D,), jnp.float32),
            pltpu.VMEM_SHARED((NS * CROW,), jnp.float32),
            pltpu.SemaphoreType.DMA,
            pltpu.SemaphoreType.DMA,
            pltpu.SemaphoreType.DMA,
            pltpu.SemaphoreType.DMA,
        ],
        compiler_params=pltpu.CompilerParams(needs_layout_passes=False),
    )
    return kern(feat2d, mask_i32)


def kernel(features, mask):
    feat2d = features.reshape(B * S, D)
    mask_i32 = mask.reshape(-1).astype(jnp.int32)
    return _pool(feat2d, mask_i32).reshape(B, D)
